# Initial kernel scaffold; baseline (speedup 1.0000x reference)
import jax
import jax.numpy as jnp
from jax.experimental import pallas as pl


def kernel(output, target, segments):
    raise NotImplementedError("write your pallas kernel here")


# speedup vs baseline: 44.1158x; 44.1158x over previous
import jax
import jax.numpy as jnp
from jax import lax
from jax.experimental import pallas as pl
from jax.experimental.pallas import tpu as pltpu
from jax.experimental.pallas import tpu_sc as plsc

NUM_CLASSES = 21
V = 1024
NPIX = 512 * 512
NC = 2
NS = 16
NW = NC * NS
PPW = NPIX // NW
GROUPS = PPW // 16
NBINS = 64


def _argmax_body(out_ref, lab_ref):
    x = out_ref[...]
    lab_ref[...] = jnp.argmax(x, axis=1).astype(jnp.int32)


def _sc_hist_body(lab_hbm, seg_hbm, tgt_hbm, out_hbm,
                  lab_v, seg_v, tgt_v, hist_v, sem1, sem2):
    cid = lax.axis_index("c")
    sid = lax.axis_index("s")
    wid = sid * NC + cid
    base = wid * PPW
    cp1 = pltpu.async_copy(seg_hbm.at[pl.ds(base, PPW)], seg_v, sem1)
    cp2 = pltpu.async_copy(tgt_hbm.at[pl.ds(base, PPW)], tgt_v, sem2)
    pltpu.sync_copy(lab_hbm, lab_v)
    zeros = jnp.zeros((16,), jnp.int32)
    for b in range(NBINS):
        hist_v[b, :] = zeros
    cp1.wait()
    cp2.wait()

    lanes = lax.iota(jnp.int32, 16)
    ones = jnp.ones((16,), jnp.int32)
    c21 = jnp.full((16,), 21, jnp.int32)
    c42 = jnp.full((16,), 42, jnp.int32)
    c63 = jnp.full((16,), 63, jnp.int32)

    def body(i, carry):
        off = i * 16
        seg = seg_v[pl.ds(off, 16)]
        tgt = tgt_v[pl.ds(off, 16)]
        lab = plsc.load_gather(lab_v, [seg])
        idxm = jnp.where(lab == tgt, lab + c42, c63)
        plsc.addupdate_scatter(hist_v, [lab, lanes], ones)
        plsc.addupdate_scatter(hist_v, [tgt + c21, lanes], ones)
        plsc.addupdate_scatter(hist_v, [idxm, lanes], ones)
        return carry

    lax.fori_loop(0, GROUPS, body, 0)
    pltpu.sync_copy(hist_v, out_hbm.at[wid])


def _fin_body(hist_ref, out_ref):
    h = hist_ref[...].astype(jnp.float32)
    tot = jnp.sum(jnp.sum(h, axis=2), axis=0, keepdims=True)
    o = tot[:, 0:NUM_CLASSES]
    t = tot[:, NUM_CLASSES:2 * NUM_CLASSES]
    m = tot[:, 2 * NUM_CLASSES:3 * NUM_CLASSES]
    score = (2.0 * m) / (o + t + 1e-10)
    out_ref[0, 0] = 1.0 - jnp.sum(score) / NUM_CLASSES


_sc_hist = pl.kernel(
    _sc_hist_body,
    out_type=jax.ShapeDtypeStruct((NW, NBINS, 16), jnp.int32),
    mesh=plsc.VectorSubcoreMesh(core_axis_name="c", subcore_axis_name="s"),
    compiler_params=pltpu.CompilerParams(needs_layout_passes=False),
    scratch_types=[
        pltpu.VMEM((V,), jnp.int32),
        pltpu.VMEM((PPW,), jnp.int32),
        pltpu.VMEM((PPW,), jnp.int32),
        pltpu.VMEM((NBINS, 16), jnp.int32),
        pltpu.SemaphoreType.DMA,
        pltpu.SemaphoreType.DMA,
    ],
)


def kernel(output, target, segments):
    labels = pl.pallas_call(
        _argmax_body,
        out_shape=jax.ShapeDtypeStruct((V,), jnp.int32),
    )(output)
    hist = _sc_hist(labels, segments.reshape(-1), target.reshape(-1))
    loss = pl.pallas_call(
        _fin_body,
        out_shape=jax.ShapeDtypeStruct((1, 1), jnp.float32),
        out_specs=pl.BlockSpec(memory_space=pltpu.SMEM),
    )(hist)
    return loss[0, 0]
